# baseline (speedup 1.0000x reference)
import functools

import jax
import jax.numpy as jnp
from jax import lax
from jax.experimental import pallas as pl
from jax.experimental.pallas import tpu as pltpu
from jax.experimental.pallas import tpu_sc as plsc

VOCAB = 1000000
EMBED = 64
BATCH = 4096

NC = 2
NS = 16
LANES = 16
NW = NC * NS
B_PER_W = BATCH // NW


def _sc_gather(word_u, word_v, in_embed, in_bias, out_embed, out_bias):
    mesh = plsc.VectorSubcoreMesh(core_axis_name="c", subcore_axis_name="s")

    @functools.partial(
        pl.kernel,
        mesh=mesh,
        out_type=(
            jax.ShapeDtypeStruct((BATCH, EMBED), jnp.float32),
            jax.ShapeDtypeStruct((BATCH, EMBED), jnp.float32),
            jax.ShapeDtypeStruct((BATCH, 1), jnp.float32),
            jax.ShapeDtypeStruct((BATCH, 1), jnp.float32),
        ),
        scratch_types=[
            pltpu.VMEM((B_PER_W,), jnp.int32),
            pltpu.VMEM((B_PER_W,), jnp.int32),
            pltpu.SemaphoreType.DMA,
            pltpu.SemaphoreType.DMA,
        ],
    )
    def k(word_u_hbm, word_v_hbm, in_embed_hbm, in_bias_hbm, out_embed_hbm,
          out_bias_hbm, u_rows_hbm, v_rows_hbm, bu_hbm, bv_hbm,
          idx_u, idx_v, sem_e, sem_b):
        wid = lax.axis_index("s") * NC + lax.axis_index("c")
        base = wid * B_PER_W

        pltpu.sync_copy(word_u_hbm.at[pl.ds(base, B_PER_W)], idx_u)
        pltpu.sync_copy(word_v_hbm.at[pl.ds(base, B_PER_W)], idx_v)

        copies = []
        for g in range(B_PER_W // LANES):
            iu = idx_u[pl.ds(g * LANES, LANES)]
            iv = idx_v[pl.ds(g * LANES, LANES)]
            for i in range(LANES):
                r = base + g * LANES + i
                copies.append(pltpu.async_copy(
                    in_embed_hbm.at[pl.ds(iu[i], 1), :],
                    u_rows_hbm.at[pl.ds(r, 1), :], sem_e))
                copies.append(pltpu.async_copy(
                    out_embed_hbm.at[pl.ds(iv[i], 1), :],
                    v_rows_hbm.at[pl.ds(r, 1), :], sem_e))
                copies.append(pltpu.async_copy(
                    in_bias_hbm.at[pl.ds(iu[i], 1), :],
                    bu_hbm.at[pl.ds(r, 1), :], sem_b))
                copies.append(pltpu.async_copy(
                    out_bias_hbm.at[pl.ds(iv[i], 1), :],
                    bv_hbm.at[pl.ds(r, 1), :], sem_b))
        for c in copies:
            c.wait()

    return k(word_u, word_v, in_embed, in_bias, out_embed, out_bias)


def _tc_dot(u_rows, v_rows):
    def body(u_ref, v_ref, s_ref):
        s_ref[...] = jnp.sum(u_ref[...] * v_ref[...], axis=1, keepdims=True)

    return pl.pallas_call(
        body,
        out_shape=jax.ShapeDtypeStruct((BATCH, 1), jnp.float32),
    )(u_rows, v_rows)


def _tc_outer_add(bu_col, bv_col, s_row):
    BM = 256

    def body(bu_ref, bv_ref, s_ref, o_ref):
        o_ref[...] = (bu_ref[...] + bv_ref[...]) + s_ref[...]

    return pl.pallas_call(
        body,
        grid=(BATCH // BM,),
        in_specs=[
            pl.BlockSpec((BM, 1), lambda i: (i, 0)),
            pl.BlockSpec((BM, 1), lambda i: (i, 0)),
            pl.BlockSpec((1, BATCH), lambda i: (0, 0)),
        ],
        out_specs=pl.BlockSpec((BM, BATCH), lambda i: (i, 0)),
        out_shape=jax.ShapeDtypeStruct((BATCH, BATCH), jnp.float32),
    )(bu_col, bv_col, s_row)


def kernel(word_u, word_v, in_embed, in_bias, out_embed, out_bias):
    word_u = word_u.astype(jnp.int32)
    word_v = word_v.astype(jnp.int32)
    u_rows, v_rows, bu, bv = _sc_gather(word_u, word_v, in_embed, in_bias,
                                        out_embed, out_bias)
    s_col = _tc_dot(u_rows, v_rows)
    return _tc_outer_add(bu, bv, s_col.reshape(1, BATCH))


# speedup vs baseline: 5.9860x; 5.9860x over previous
import functools

import jax
import jax.numpy as jnp
from jax import lax
from jax.experimental import pallas as pl
from jax.experimental.pallas import tpu as pltpu
from jax.experimental.pallas import tpu_sc as plsc

VOCAB = 1000000
EMBED = 64
BATCH = 4096

NC = 2
NS = 16
LANES = 16
NW = NC * NS
B_PER_W = BATCH // NW
NBUF = 4


def _sc_gather(word_u, word_v, in_embed_t, in_bias_f, out_embed_t,
               out_bias_f):
    mesh = plsc.VectorSubcoreMesh(core_axis_name="c", subcore_axis_name="s")

    @functools.partial(
        pl.kernel,
        mesh=mesh,
        out_type=(
            jax.ShapeDtypeStruct((EMBED, BATCH), jnp.float32),
            jax.ShapeDtypeStruct((EMBED, BATCH), jnp.float32),
            jax.ShapeDtypeStruct((BATCH,), jnp.float32),
        ),
        scratch_types=[
            pltpu.VMEM((B_PER_W,), jnp.int32),
            pltpu.VMEM((B_PER_W,), jnp.int32),
            pltpu.VMEM((NBUF, EMBED, 128), jnp.float32),
            pltpu.VMEM((NBUF, EMBED, 128), jnp.float32),
            pltpu.VMEM_SHARED((NS, EMBED, B_PER_W), jnp.float32),
            pltpu.VMEM_SHARED((NS, EMBED, B_PER_W), jnp.float32),
            pltpu.VMEM((B_PER_W,), jnp.float32),
            pltpu.VMEM((B_PER_W,), jnp.float32),
            pltpu.VMEM((B_PER_W,), jnp.float32),
            pltpu.SemaphoreType.DMA,
            pltpu.SemaphoreType.DMA,
            pltpu.SemaphoreType.DMA,
            pltpu.SemaphoreType.DMA,
        ],
    )
    def k(word_u_hbm, word_v_hbm, in_embed_hbm, in_bias_hbm, out_embed_hbm,
          out_bias_hbm, u_cols_hbm, v_cols_hbm, b_hbm,
          idx_u, idx_v, win_u, win_v, col_u, col_v, bias_u, bias_v, b_loc,
          sem_u, sem_v, sem_bu, sem_bv):
        sid = lax.axis_index("s")
        wid = sid * NC + lax.axis_index("c")
        base = wid * B_PER_W

        pltpu.sync_copy(word_u_hbm.at[pl.ds(base, B_PER_W)], idx_u)
        pltpu.sync_copy(word_v_hbm.at[pl.ds(base, B_PER_W)], idx_v)

        cbu = pltpu.async_copy(in_bias_hbm.at[idx_u], bias_u, sem_bu)
        cbv = pltpu.async_copy(out_bias_hbm.at[idx_v], bias_v, sem_bv)

        scal_u, scal_v = [], []
        for g in range(B_PER_W // LANES):
            iu = idx_u[pl.ds(g * LANES, LANES)]
            iv = idx_v[pl.ds(g * LANES, LANES)]
            for i in range(LANES):
                scal_u.append(iu[i])
                scal_v.append(iv[i])

        def fire(k_el):
            slot = k_el % NBUF
            au = pl.multiple_of((scal_u[k_el] // 128) * 128, 128)
            av = pl.multiple_of((scal_v[k_el] // 128) * 128, 128)
            cu = pltpu.async_copy(
                in_embed_hbm.at[:, pl.ds(au, 128)], win_u.at[slot], sem_u)
            cv = pltpu.async_copy(
                out_embed_hbm.at[:, pl.ds(av, 128)], win_v.at[slot], sem_v)
            return (cu, cv)

        inflight = [fire(k_el) for k_el in range(NBUF)]
        for k_el in range(B_PER_W):
            slot = k_el % NBUF
            cu, cv = inflight[slot]
            cu.wait()
            cv.wait()
            lu = lax.rem(scal_u[k_el], 128)
            lv = lax.rem(scal_v[k_el], 128)
            pltpu.sync_copy(win_u.at[slot, :, pl.ds(lu, 1)],
                            col_u.at[sid, :, pl.ds(k_el, 1)])
            pltpu.sync_copy(win_v.at[slot, :, pl.ds(lv, 1)],
                            col_v.at[sid, :, pl.ds(k_el, 1)])
            if k_el + NBUF < B_PER_W:
                inflight[slot] = fire(k_el + NBUF)

        pltpu.sync_copy(col_u.at[sid], u_cols_hbm.at[:, pl.ds(base, B_PER_W)])
        pltpu.sync_copy(col_v.at[sid], v_cols_hbm.at[:, pl.ds(base, B_PER_W)])

        cbu.wait()
        cbv.wait()
        for g in range(B_PER_W // LANES):
            sl = pl.ds(g * LANES, LANES)
            b_loc[sl] = bias_u[sl] + bias_v[sl]
        pltpu.sync_copy(b_loc, b_hbm.at[pl.ds(base, B_PER_W)])

    return k(word_u, word_v, in_embed_t, in_bias_f, out_embed_t,
             out_bias_f)


def _tc_dot(u_cols, v_cols):
    def body(u_ref, v_ref, s_ref):
        s_ref[...] = jnp.sum(u_ref[...] * v_ref[...], axis=0, keepdims=True)

    return pl.pallas_call(
        body,
        out_shape=jax.ShapeDtypeStruct((1, BATCH), jnp.float32),
    )(u_cols, v_cols)


def _tc_outer_add(b_col, s_row):
    BM = 256

    def body(b_ref, s_ref, o_ref):
        o_ref[...] = b_ref[...] + s_ref[...]

    return pl.pallas_call(
        body,
        grid=(BATCH // BM,),
        in_specs=[
            pl.BlockSpec((BM, 1), lambda i: (i, 0)),
            pl.BlockSpec((1, BATCH), lambda i: (0, 0)),
        ],
        out_specs=pl.BlockSpec((BM, BATCH), lambda i: (i, 0)),
        out_shape=jax.ShapeDtypeStruct((BATCH, BATCH), jnp.float32),
    )(b_col, s_row)


def kernel(word_u, word_v, in_embed, in_bias, out_embed, out_bias):
    word_u = word_u.astype(jnp.int32)
    word_v = word_v.astype(jnp.int32)
    in_embed_t = jnp.swapaxes(in_embed, 0, 1)
    out_embed_t = jnp.swapaxes(out_embed, 0, 1)
    in_bias_f = jnp.swapaxes(in_bias, 0, 1).reshape(VOCAB)
    out_bias_f = jnp.swapaxes(out_bias, 0, 1).reshape(VOCAB)
    u_cols, v_cols, b = _sc_gather(word_u, word_v, in_embed_t, in_bias_f,
                                   out_embed_t, out_bias_f)
    s_row = _tc_dot(u_cols, v_cols)
    return _tc_outer_add(b.reshape(BATCH, 1), s_row)


# speedup vs baseline: 8.7874x; 1.4680x over previous
import functools

import jax
import jax.numpy as jnp
from jax import lax
from jax.experimental import pallas as pl
from jax.experimental.pallas import tpu as pltpu
from jax.experimental.pallas import tpu_sc as plsc

VOCAB = 1000000
EMBED = 64
BATCH = 4096

NC = 2
NS = 16
LANES = 16
NW = NC * NS
B_PER_W = BATCH // NW
NBUF = 6
NGRP = B_PER_W // LANES


def _sc_gather(word_u, word_v, in_embed_t, in_bias_t, out_embed_t,
               out_bias_t):
    mesh = plsc.VectorSubcoreMesh(core_axis_name="c", subcore_axis_name="s")

    @functools.partial(
        pl.kernel,
        mesh=mesh,
        out_type=(
            jax.ShapeDtypeStruct((EMBED, BATCH), jnp.float32),
            jax.ShapeDtypeStruct((EMBED, BATCH), jnp.float32),
            jax.ShapeDtypeStruct((1, BATCH), jnp.float32),
            jax.ShapeDtypeStruct((1, BATCH), jnp.float32),
        ),
        scratch_types=[
            pltpu.VMEM((B_PER_W,), jnp.int32),
            pltpu.VMEM((B_PER_W,), jnp.int32),
            pltpu.VMEM((NBUF, EMBED, 128), jnp.float32),
            pltpu.VMEM((NBUF, EMBED, 128), jnp.float32),
            pltpu.VMEM((NBUF, 1, 128), jnp.float32),
            pltpu.VMEM((NBUF, 1, 128), jnp.float32),
            pltpu.VMEM_SHARED((NS, EMBED, B_PER_W), jnp.float32),
            pltpu.VMEM_SHARED((NS, EMBED, B_PER_W), jnp.float32),
            pltpu.VMEM_SHARED((NS, 1, B_PER_W), jnp.float32),
            pltpu.VMEM_SHARED((NS, 1, B_PER_W), jnp.float32),
            pltpu.SemaphoreType.DMA,
            pltpu.SemaphoreType.DMA,
        ],
    )
    def k(word_u_hbm, word_v_hbm, in_embed_hbm, in_bias_hbm, out_embed_hbm,
          out_bias_hbm, u_cols_hbm, v_cols_hbm, bu_hbm, bv_hbm,
          idx_u, idx_v, win_u, win_v, win_bu, win_bv,
          col_u, col_v, col_bu, col_bv, sem_w, sem_c):
        sid = lax.axis_index("s")
        wid = sid * NC + lax.axis_index("c")
        base = wid * B_PER_W

        pltpu.sync_copy(word_u_hbm.at[pl.ds(base, B_PER_W)], idx_u)
        pltpu.sync_copy(word_v_hbm.at[pl.ds(base, B_PER_W)], idx_v)

        def group(g, carry):
            gb = pl.multiple_of(g * LANES, LANES)
            iu = idx_u[pl.ds(gb, LANES)]
            iv = idx_v[pl.ds(gb, LANES)]
            su = [iu[i] for i in range(LANES)]
            sv = [iv[i] for i in range(LANES)]

            def fire(i):
                slot = i % NBUF
                au = pl.multiple_of((su[i] // 128) * 128, 128)
                av = pl.multiple_of((sv[i] // 128) * 128, 128)
                return (
                    pltpu.async_copy(in_embed_hbm.at[:, pl.ds(au, 128)],
                                     win_u.at[slot], sem_w),
                    pltpu.async_copy(out_embed_hbm.at[:, pl.ds(av, 128)],
                                     win_v.at[slot], sem_w),
                    pltpu.async_copy(in_bias_hbm.at[:, pl.ds(au, 128)],
                                     win_bu.at[slot], sem_w),
                    pltpu.async_copy(out_bias_hbm.at[:, pl.ds(av, 128)],
                                     win_bv.at[slot], sem_w),
                )

            wfly = {i: fire(i) for i in range(min(NBUF - 2, LANES))}
            cfly = {}
            for i in range(LANES):
                slot = i % NBUF
                for c in wfly.pop(i):
                    c.wait()
                lu = lax.rem(su[i], 128)
                lv = lax.rem(sv[i], 128)
                kc = gb + i
                cfly[i] = (
                    pltpu.async_copy(win_u.at[slot, :, pl.ds(lu, 1)],
                                     col_u.at[sid, :, pl.ds(kc, 1)], sem_c),
                    pltpu.async_copy(win_v.at[slot, :, pl.ds(lv, 1)],
                                     col_v.at[sid, :, pl.ds(kc, 1)], sem_c),
                    pltpu.async_copy(win_bu.at[slot, :, pl.ds(lu, 1)],
                                     col_bu.at[sid, :, pl.ds(kc, 1)], sem_c),
                    pltpu.async_copy(win_bv.at[slot, :, pl.ds(lv, 1)],
                                     col_bv.at[sid, :, pl.ds(kc, 1)], sem_c),
                )
                nxt = i + NBUF - 2
                if nxt < LANES:
                    if i >= 2:
                        for c in cfly.pop(i - 2):
                            c.wait()
                    wfly[nxt] = fire(nxt)
            for i in sorted(cfly):
                for c in cfly[i]:
                    c.wait()
            return carry

        lax.fori_loop(0, NGRP, group, 0)

        pltpu.sync_copy(col_u.at[sid], u_cols_hbm.at[:, pl.ds(base, B_PER_W)])
        pltpu.sync_copy(col_v.at[sid], v_cols_hbm.at[:, pl.ds(base, B_PER_W)])
        pltpu.sync_copy(col_bu.at[sid], bu_hbm.at[:, pl.ds(base, B_PER_W)])
        pltpu.sync_copy(col_bv.at[sid], bv_hbm.at[:, pl.ds(base, B_PER_W)])

    return k(word_u, word_v, in_embed_t, in_bias_t, out_embed_t, out_bias_t)


def _tc_compute(u_cols, v_cols, bu_col, bv_col):
    BM = 256

    def body(u_ref, v_ref, bu_ref, bv_ref, o_ref, s_ref):
        @pl.when(pl.program_id(0) == 0)
        def _():
            s_ref[...] = jnp.sum(u_ref[...] * v_ref[...], axis=0,
                                 keepdims=True)

        o_ref[...] = (bu_ref[...] + bv_ref[...]) + s_ref[...]

    return pl.pallas_call(
        body,
        grid=(BATCH // BM,),
        in_specs=[
            pl.BlockSpec((EMBED, BATCH), lambda i: (0, 0)),
            pl.BlockSpec((EMBED, BATCH), lambda i: (0, 0)),
            pl.BlockSpec((BM, 1), lambda i: (i, 0)),
            pl.BlockSpec((BM, 1), lambda i: (i, 0)),
        ],
        out_specs=pl.BlockSpec((BM, BATCH), lambda i: (i, 0)),
        out_shape=jax.ShapeDtypeStruct((BATCH, BATCH), jnp.float32),
        scratch_shapes=[pltpu.VMEM((1, BATCH), jnp.float32)],
    )(u_cols, v_cols, bu_col, bv_col)


def kernel(word_u, word_v, in_embed, in_bias, out_embed, out_bias):
    word_u = word_u.astype(jnp.int32)
    word_v = word_v.astype(jnp.int32)
    in_embed_t = jnp.swapaxes(in_embed, 0, 1)
    out_embed_t = jnp.swapaxes(out_embed, 0, 1)
    in_bias_t = jnp.swapaxes(in_bias, 0, 1)
    out_bias_t = jnp.swapaxes(out_bias, 0, 1)
    u_cols, v_cols, bu, bv = _sc_gather(word_u, word_v, in_embed_t,
                                        in_bias_t, out_embed_t, out_bias_t)
    return _tc_compute(u_cols, v_cols, bu.reshape(BATCH, 1),
                       bv.reshape(BATCH, 1))


# speedup vs baseline: 8.8709x; 1.0095x over previous
import functools

import jax
import jax.numpy as jnp
from jax import lax
from jax.experimental import pallas as pl
from jax.experimental.pallas import tpu as pltpu
from jax.experimental.pallas import tpu_sc as plsc

VOCAB = 1000000
EMBED = 64
BATCH = 4096

NC = 2
NS = 16
LANES = 16
NW = NC * NS
B_PER_W = BATCH // NW
NBUF = 6
NGRP = B_PER_W // LANES


def _sc_gather(word_u, word_v, in_embed_t, in_bias_t, out_embed_t,
               out_bias_t):
    mesh = plsc.VectorSubcoreMesh(core_axis_name="c", subcore_axis_name="s")

    @functools.partial(
        pl.kernel,
        mesh=mesh,
        out_type=(
            jax.ShapeDtypeStruct((EMBED, BATCH), jnp.float32),
            jax.ShapeDtypeStruct((EMBED, BATCH), jnp.float32),
            jax.ShapeDtypeStruct((1, BATCH), jnp.float32),
            jax.ShapeDtypeStruct((1, BATCH), jnp.float32),
        ),
        scratch_types=[
            pltpu.VMEM((B_PER_W,), jnp.int32),
            pltpu.VMEM((B_PER_W,), jnp.int32),
            pltpu.VMEM((NBUF, EMBED, 128), jnp.float32),
            pltpu.VMEM((NBUF, EMBED, 128), jnp.float32),
            pltpu.VMEM((NBUF, 1, 128), jnp.float32),
            pltpu.VMEM((NBUF, 1, 128), jnp.float32),
            pltpu.VMEM_SHARED((NS, EMBED, B_PER_W), jnp.float32),
            pltpu.VMEM_SHARED((NS, EMBED, B_PER_W), jnp.float32),
            pltpu.VMEM_SHARED((NS, 1, B_PER_W), jnp.float32),
            pltpu.VMEM_SHARED((NS, 1, B_PER_W), jnp.float32),
            pltpu.SemaphoreType.DMA,
            pltpu.SemaphoreType.DMA,
        ],
    )
    def k(word_u_hbm, word_v_hbm, in_embed_hbm, in_bias_hbm, out_embed_hbm,
          out_bias_hbm, u_cols_hbm, v_cols_hbm, bu_hbm, bv_hbm,
          idx_u, idx_v, win_u, win_v, win_bu, win_bv,
          col_u, col_v, col_bu, col_bv, sem_w, sem_c):
        sid = lax.axis_index("s")
        wid = sid * NC + lax.axis_index("c")
        base = wid * B_PER_W

        pltpu.sync_copy(word_u_hbm.at[pl.ds(base, B_PER_W)], idx_u)
        pltpu.sync_copy(word_v_hbm.at[pl.ds(base, B_PER_W)], idx_v)

        def group(g, carry):
            gb = pl.multiple_of(g * LANES, LANES)
            iu = idx_u[pl.ds(gb, LANES)]
            iv = idx_v[pl.ds(gb, LANES)]
            su = [iu[i] for i in range(LANES)]
            sv = [iv[i] for i in range(LANES)]

            def fire(i):
                slot = i % NBUF
                au = pl.multiple_of((su[i] // 128) * 128, 128)
                av = pl.multiple_of((sv[i] // 128) * 128, 128)
                return (
                    pltpu.async_copy(in_embed_hbm.at[:, pl.ds(au, 128)],
                                     win_u.at[slot], sem_w),
                    pltpu.async_copy(out_embed_hbm.at[:, pl.ds(av, 128)],
                                     win_v.at[slot], sem_w),
                    pltpu.async_copy(in_bias_hbm.at[:, pl.ds(au, 128)],
                                     win_bu.at[slot], sem_w),
                    pltpu.async_copy(out_bias_hbm.at[:, pl.ds(av, 128)],
                                     win_bv.at[slot], sem_w),
                )

            wfly = {i: fire(i) for i in range(min(NBUF - 2, LANES))}
            cfly = {}
            for i in range(LANES):
                slot = i % NBUF
                for c in wfly.pop(i):
                    c.wait()
                lu = lax.rem(su[i], 128)
                lv = lax.rem(sv[i], 128)
                kc = gb + i
                cfly[i] = (
                    pltpu.async_copy(win_u.at[slot, :, pl.ds(lu, 1)],
                                     col_u.at[sid, :, pl.ds(kc, 1)], sem_c),
                    pltpu.async_copy(win_v.at[slot, :, pl.ds(lv, 1)],
                                     col_v.at[sid, :, pl.ds(kc, 1)], sem_c),
                    pltpu.async_copy(win_bu.at[slot, :, pl.ds(lu, 1)],
                                     col_bu.at[sid, :, pl.ds(kc, 1)], sem_c),
                    pltpu.async_copy(win_bv.at[slot, :, pl.ds(lv, 1)],
                                     col_bv.at[sid, :, pl.ds(kc, 1)], sem_c),
                )
                nxt = i + NBUF - 2
                if nxt < LANES:
                    if i >= 2:
                        for c in cfly.pop(i - 2):
                            c.wait()
                    wfly[nxt] = fire(nxt)
            for i in sorted(cfly):
                for c in cfly[i]:
                    c.wait()
            return carry

        lax.fori_loop(0, NGRP, group, 0)

        pltpu.sync_copy(col_u.at[sid], u_cols_hbm.at[:, pl.ds(base, B_PER_W)])
        pltpu.sync_copy(col_v.at[sid], v_cols_hbm.at[:, pl.ds(base, B_PER_W)])
        pltpu.sync_copy(col_bu.at[sid], bu_hbm.at[:, pl.ds(base, B_PER_W)])
        pltpu.sync_copy(col_bv.at[sid], bv_hbm.at[:, pl.ds(base, B_PER_W)])

    return k(word_u, word_v, in_embed_t, in_bias_t, out_embed_t, out_bias_t)


def _tc_compute(u_cols, v_cols, bu_col, bv_col):
    BM = 512

    def body(u_ref, v_ref, bu_ref, bv_ref, o_ref, s_ref):
        @pl.when(pl.program_id(0) == 0)
        def _():
            s_ref[...] = jnp.sum(u_ref[...] * v_ref[...], axis=0,
                                 keepdims=True)

        o_ref[...] = (bu_ref[...] + bv_ref[...]) + s_ref[...]

    return pl.pallas_call(
        body,
        grid=(BATCH // BM,),
        in_specs=[
            pl.BlockSpec((EMBED, BATCH), lambda i: (0, 0)),
            pl.BlockSpec((EMBED, BATCH), lambda i: (0, 0)),
            pl.BlockSpec((BM, 1), lambda i: (i, 0)),
            pl.BlockSpec((BM, 1), lambda i: (i, 0)),
        ],
        out_specs=pl.BlockSpec((BM, BATCH), lambda i: (i, 0)),
        out_shape=jax.ShapeDtypeStruct((BATCH, BATCH), jnp.float32),
        scratch_shapes=[pltpu.VMEM((1, BATCH), jnp.float32)],
    )(u_cols, v_cols, bu_col, bv_col)


def kernel(word_u, word_v, in_embed, in_bias, out_embed, out_bias):
    word_u = word_u.astype(jnp.int32)
    word_v = word_v.astype(jnp.int32)
    in_embed_t = jnp.swapaxes(in_embed, 0, 1)
    out_embed_t = jnp.swapaxes(out_embed, 0, 1)
    in_bias_t = jnp.swapaxes(in_bias, 0, 1)
    out_bias_t = jnp.swapaxes(out_bias, 0, 1)
    u_cols, v_cols, bu, bv = _sc_gather(word_u, word_v, in_embed_t,
                                        in_bias_t, out_embed_t, out_bias_t)
    return _tc_compute(u_cols, v_cols, bu.reshape(BATCH, 1),
                       bv.reshape(BATCH, 1))


# speedup vs baseline: 9.0537x; 1.0206x over previous
import functools

import jax
import jax.numpy as jnp
from jax import lax
from jax.experimental import pallas as pl
from jax.experimental.pallas import tpu as pltpu
from jax.experimental.pallas import tpu_sc as plsc

VOCAB = 1000000
EMBED = 64
BATCH = 4096

NC = 2
NS = 16
LANES = 16
NW = NC * NS
B_PER_W = BATCH // NW
NBUF = 6
GRP = 32
NGRP = B_PER_W // GRP


def _sc_gather(word_u, word_v, in_embed_t, in_bias_t, out_embed_t,
               out_bias_t):
    mesh = plsc.VectorSubcoreMesh(core_axis_name="c", subcore_axis_name="s")

    @functools.partial(
        pl.kernel,
        mesh=mesh,
        out_type=(
            jax.ShapeDtypeStruct((EMBED, BATCH), jnp.float32),
            jax.ShapeDtypeStruct((EMBED, BATCH), jnp.float32),
            jax.ShapeDtypeStruct((1, BATCH), jnp.float32),
            jax.ShapeDtypeStruct((1, BATCH), jnp.float32),
        ),
        scratch_types=[
            pltpu.VMEM((B_PER_W,), jnp.int32),
            pltpu.VMEM((B_PER_W,), jnp.int32),
            pltpu.VMEM((NBUF, EMBED, 128), jnp.float32),
            pltpu.VMEM((NBUF, EMBED, 128), jnp.float32),
            pltpu.VMEM((NBUF, 1, 128), jnp.float32),
            pltpu.VMEM((NBUF, 1, 128), jnp.float32),
            pltpu.VMEM_SHARED((NS, EMBED, B_PER_W), jnp.float32),
            pltpu.VMEM_SHARED((NS, EMBED, B_PER_W), jnp.float32),
            pltpu.VMEM_SHARED((NS, 1, B_PER_W), jnp.float32),
            pltpu.VMEM_SHARED((NS, 1, B_PER_W), jnp.float32),
            pltpu.SemaphoreType.DMA,
            pltpu.SemaphoreType.DMA,
        ],
    )
    def k(word_u_hbm, word_v_hbm, in_embed_hbm, in_bias_hbm, out_embed_hbm,
          out_bias_hbm, u_cols_hbm, v_cols_hbm, bu_hbm, bv_hbm,
          idx_u, idx_v, win_u, win_v, win_bu, win_bv,
          col_u, col_v, col_bu, col_bv, sem_w, sem_c):
        sid = lax.axis_index("s")
        wid = sid * NC + lax.axis_index("c")
        base = wid * B_PER_W

        pltpu.sync_copy(word_u_hbm.at[pl.ds(base, B_PER_W)], idx_u)
        pltpu.sync_copy(word_v_hbm.at[pl.ds(base, B_PER_W)], idx_v)

        def group(g, carry):
            gb = pl.multiple_of(g * GRP, GRP)
            su, sv = [], []
            for h in range(GRP // LANES):
                iu = idx_u[pl.ds(gb + h * LANES, LANES)]
                iv = idx_v[pl.ds(gb + h * LANES, LANES)]
                su += [iu[i] for i in range(LANES)]
                sv += [iv[i] for i in range(LANES)]

            def fire(i):
                slot = i % NBUF
                au = pl.multiple_of((su[i] // 128) * 128, 128)
                av = pl.multiple_of((sv[i] // 128) * 128, 128)
                return (
                    pltpu.async_copy(in_embed_hbm.at[:, pl.ds(au, 128)],
                                     win_u.at[slot], sem_w),
                    pltpu.async_copy(out_embed_hbm.at[:, pl.ds(av, 128)],
                                     win_v.at[slot], sem_w),
                    pltpu.async_copy(in_bias_hbm.at[:, pl.ds(au, 128)],
                                     win_bu.at[slot], sem_w),
                    pltpu.async_copy(out_bias_hbm.at[:, pl.ds(av, 128)],
                                     win_bv.at[slot], sem_w),
                )

            wfly = {i: fire(i) for i in range(min(NBUF - 2, GRP))}
            cfly = {}
            for i in range(GRP):
                slot = i % NBUF
                for c in wfly.pop(i):
                    c.wait()
                lu = lax.rem(su[i], 128)
                lv = lax.rem(sv[i], 128)
                kc = gb + i
                cfly[i] = (
                    pltpu.async_copy(win_u.at[slot, :, pl.ds(lu, 1)],
                                     col_u.at[sid, :, pl.ds(kc, 1)], sem_c),
                    pltpu.async_copy(win_v.at[slot, :, pl.ds(lv, 1)],
                                     col_v.at[sid, :, pl.ds(kc, 1)], sem_c),
                    pltpu.async_copy(win_bu.at[slot, :, pl.ds(lu, 1)],
                                     col_bu.at[sid, :, pl.ds(kc, 1)], sem_c),
                    pltpu.async_copy(win_bv.at[slot, :, pl.ds(lv, 1)],
                                     col_bv.at[sid, :, pl.ds(kc, 1)], sem_c),
                )
                nxt = i + NBUF - 2
                if nxt < GRP:
                    if i >= 2:
                        for c in cfly.pop(i - 2):
                            c.wait()
                    wfly[nxt] = fire(nxt)
            for i in sorted(cfly):
                for c in cfly[i]:
                    c.wait()
            return carry

        lax.fori_loop(0, NGRP, group, 0)

        pltpu.sync_copy(col_u.at[sid], u_cols_hbm.at[:, pl.ds(base, B_PER_W)])
        pltpu.sync_copy(col_v.at[sid], v_cols_hbm.at[:, pl.ds(base, B_PER_W)])
        pltpu.sync_copy(col_bu.at[sid], bu_hbm.at[:, pl.ds(base, B_PER_W)])
        pltpu.sync_copy(col_bv.at[sid], bv_hbm.at[:, pl.ds(base, B_PER_W)])

    return k(word_u, word_v, in_embed_t, in_bias_t, out_embed_t, out_bias_t)


def _tc_compute(u_cols, v_cols, bu_col, bv_col):
    BM = 1024

    def body(u_ref, v_ref, bu_ref, bv_ref, o_ref, s_ref):
        @pl.when(pl.program_id(0) == 0)
        def _():
            s_ref[...] = jnp.sum(u_ref[...] * v_ref[...], axis=0,
                                 keepdims=True)

        o_ref[...] = (bu_ref[...] + bv_ref[...]) + s_ref[...]

    return pl.pallas_call(
        body,
        grid=(BATCH // BM,),
        in_specs=[
            pl.BlockSpec((EMBED, BATCH), lambda i: (0, 0)),
            pl.BlockSpec((EMBED, BATCH), lambda i: (0, 0)),
            pl.BlockSpec((BM, 1), lambda i: (i, 0)),
            pl.BlockSpec((BM, 1), lambda i: (i, 0)),
        ],
        out_specs=pl.BlockSpec((BM, BATCH), lambda i: (i, 0)),
        out_shape=jax.ShapeDtypeStruct((BATCH, BATCH), jnp.float32),
        scratch_shapes=[pltpu.VMEM((1, BATCH), jnp.float32)],
    )(u_cols, v_cols, bu_col, bv_col)


def kernel(word_u, word_v, in_embed, in_bias, out_embed, out_bias):
    word_u = word_u.astype(jnp.int32)
    word_v = word_v.astype(jnp.int32)
    in_embed_t = jnp.swapaxes(in_embed, 0, 1)
    out_embed_t = jnp.swapaxes(out_embed, 0, 1)
    in_bias_t = jnp.swapaxes(in_bias, 0, 1)
    out_bias_t = jnp.swapaxes(out_bias, 0, 1)
    u_cols, v_cols, bu, bv = _sc_gather(word_u, word_v, in_embed_t,
                                        in_bias_t, out_embed_t, out_bias_t)
    return _tc_compute(u_cols, v_cols, bu.reshape(BATCH, 1),
                       bv.reshape(BATCH, 1))


# speedup vs baseline: 9.4224x; 1.0407x over previous
import functools

import jax
import jax.numpy as jnp
from jax import lax
from jax.experimental import pallas as pl
from jax.experimental.pallas import tpu as pltpu
from jax.experimental.pallas import tpu_sc as plsc

VOCAB = 1000000
EMBED = 64
BATCH = 4096

NC = 2
NS = 16
LANES = 16
NW = NC * NS
B_PER_W = BATCH // NW
NBUF = 6
GRP = 32
NGRP = B_PER_W // GRP


def _sc_gather(word_u, word_v, in_embed_t, in_bias_t, out_embed_t,
               out_bias_t):
    mesh = plsc.VectorSubcoreMesh(core_axis_name="c", subcore_axis_name="s")

    @functools.partial(
        pl.kernel,
        mesh=mesh,
        out_type=(
            jax.ShapeDtypeStruct((EMBED, BATCH), jnp.float32),
            jax.ShapeDtypeStruct((EMBED, BATCH), jnp.float32),
            jax.ShapeDtypeStruct((1, BATCH), jnp.float32),
            jax.ShapeDtypeStruct((1, BATCH), jnp.float32),
        ),
        scratch_types=[
            pltpu.VMEM((B_PER_W,), jnp.int32),
            pltpu.VMEM((B_PER_W,), jnp.int32),
            pltpu.VMEM((NBUF, EMBED, 128), jnp.float32),
            pltpu.VMEM((NBUF, EMBED, 128), jnp.float32),
            pltpu.VMEM((NBUF, 1, 128), jnp.float32),
            pltpu.VMEM((NBUF, 1, 128), jnp.float32),
            pltpu.VMEM_SHARED((NS, EMBED, B_PER_W), jnp.float32),
            pltpu.VMEM_SHARED((NS, EMBED, B_PER_W), jnp.float32),
            pltpu.VMEM_SHARED((NS, 1, B_PER_W), jnp.float32),
            pltpu.VMEM_SHARED((NS, 1, B_PER_W), jnp.float32),
            pltpu.SemaphoreType.DMA,
            pltpu.SemaphoreType.DMA,
        ],
    )
    def k(word_u_hbm, word_v_hbm, in_embed_hbm, in_bias_hbm, out_embed_hbm,
          out_bias_hbm, u_cols_hbm, v_cols_hbm, bu_hbm, bv_hbm,
          idx_u, idx_v, win_u, win_v, win_bu, win_bv,
          col_u, col_v, col_bu, col_bv, sem_w, sem_c):
        sid = lax.axis_index("s")
        wid = sid * NC + lax.axis_index("c")
        base = wid * B_PER_W

        pltpu.sync_copy(word_u_hbm.at[pl.ds(base, B_PER_W)], idx_u)
        pltpu.sync_copy(word_v_hbm.at[pl.ds(base, B_PER_W)], idx_v)

        def group(g, carry):
            gb = pl.multiple_of(g * GRP, GRP)
            su, sv = [], []
            for h in range(GRP // LANES):
                iu = idx_u[pl.ds(gb + h * LANES, LANES)]
                iv = idx_v[pl.ds(gb + h * LANES, LANES)]
                su += [iu[i] for i in range(LANES)]
                sv += [iv[i] for i in range(LANES)]

            def fire(i):
                slot = i % NBUF
                au = pl.multiple_of((su[i] // 128) * 128, 128)
                av = pl.multiple_of((sv[i] // 128) * 128, 128)
                return (
                    pltpu.async_copy(in_embed_hbm.at[:, pl.ds(au, 128)],
                                     win_u.at[slot], sem_w),
                    pltpu.async_copy(out_embed_hbm.at[:, pl.ds(av, 128)],
                                     win_v.at[slot], sem_w),
                    pltpu.async_copy(in_bias_hbm.at[:, pl.ds(au, 128)],
                                     win_bu.at[slot], sem_w),
                    pltpu.async_copy(out_bias_hbm.at[:, pl.ds(av, 128)],
                                     win_bv.at[slot], sem_w),
                )

            wfly = {i: fire(i) for i in range(min(NBUF - 2, GRP))}
            cfly = {}
            for i in range(GRP):
                slot = i % NBUF
                for c in wfly.pop(i):
                    c.wait()
                lu = lax.rem(su[i], 128)
                lv = lax.rem(sv[i], 128)
                kc = gb + i
                cfly[i] = (
                    pltpu.async_copy(win_u.at[slot, :, pl.ds(lu, 1)],
                                     col_u.at[sid, :, pl.ds(kc, 1)], sem_c),
                    pltpu.async_copy(win_v.at[slot, :, pl.ds(lv, 1)],
                                     col_v.at[sid, :, pl.ds(kc, 1)], sem_c),
                    pltpu.async_copy(win_bu.at[slot, :, pl.ds(lu, 1)],
                                     col_bu.at[sid, :, pl.ds(kc, 1)], sem_c),
                    pltpu.async_copy(win_bv.at[slot, :, pl.ds(lv, 1)],
                                     col_bv.at[sid, :, pl.ds(kc, 1)], sem_c),
                )
                nxt = i + NBUF - 2
                if nxt < GRP:
                    if i >= 2:
                        for c in cfly.pop(i - 2):
                            c.wait()
                    wfly[nxt] = fire(nxt)
            for i in sorted(cfly):
                for c in cfly[i]:
                    c.wait()
            return carry

        lax.fori_loop(0, NGRP, group, 0)

        pltpu.sync_copy(col_u.at[sid], u_cols_hbm.at[:, pl.ds(base, B_PER_W)])
        pltpu.sync_copy(col_v.at[sid], v_cols_hbm.at[:, pl.ds(base, B_PER_W)])
        pltpu.sync_copy(col_bu.at[sid], bu_hbm.at[:, pl.ds(base, B_PER_W)])
        pltpu.sync_copy(col_bv.at[sid], bv_hbm.at[:, pl.ds(base, B_PER_W)])

    return k(word_u, word_v, in_embed_t, in_bias_t, out_embed_t, out_bias_t)


def _tc_compute(u_cols, v_cols, bu_col, bv_col):
    BM = 1024

    def body(u_ref, v_ref, bu_ref, bv_ref, o_ref, s_ref):
        @pl.when(pl.program_id(0) == 0)
        def _():
            s_ref[...] = jnp.sum(u_ref[...] * v_ref[...], axis=0,
                                 keepdims=True)

        b_col = jnp.transpose(bu_ref[...] + bv_ref[...], (1, 0))
        o_ref[...] = b_col + s_ref[...]

    return pl.pallas_call(
        body,
        grid=(BATCH // BM,),
        in_specs=[
            pl.BlockSpec((EMBED, BATCH), lambda i: (0, 0)),
            pl.BlockSpec((EMBED, BATCH), lambda i: (0, 0)),
            pl.BlockSpec((1, BM), lambda i: (0, i)),
            pl.BlockSpec((1, BM), lambda i: (0, i)),
        ],
        out_specs=pl.BlockSpec((BM, BATCH), lambda i: (i, 0)),
        out_shape=jax.ShapeDtypeStruct((BATCH, BATCH), jnp.float32),
        scratch_shapes=[pltpu.VMEM((1, BATCH), jnp.float32)],
    )(u_cols, v_cols, bu_col, bv_col)


def kernel(word_u, word_v, in_embed, in_bias, out_embed, out_bias):
    word_u = word_u.astype(jnp.int32)
    word_v = word_v.astype(jnp.int32)
    in_embed_t = jnp.swapaxes(in_embed, 0, 1)
    out_embed_t = jnp.swapaxes(out_embed, 0, 1)
    in_bias_t = jnp.swapaxes(in_bias, 0, 1)
    out_bias_t = jnp.swapaxes(out_bias, 0, 1)
    u_cols, v_cols, bu, bv = _sc_gather(word_u, word_v, in_embed_t,
                                        in_bias_t, out_embed_t, out_bias_t)
    return _tc_compute(u_cols, v_cols, bu, bv)
